# baseline (speedup 1.0000x reference)
import functools

import jax
import jax.numpy as jnp
from jax import lax
from jax.experimental import pallas as pl
from jax.experimental.pallas import tpu as pltpu
from jax.experimental.pallas import tpu_sc as plsc

_D = 64
_NW = 32
_V = 100000
_RW = 3200
_RCH = 160
_NCH_W = _RW // _RCH
_XR_W = 128
_SCH = 2
_NCH_G = 50 // _SCH


def _mesh():
    return plsc.VectorSubcoreMesh(core_axis_name="c", subcore_axis_name="s")


@jax.jit
def _widen(weight):
    @functools.partial(
        pl.kernel,
        out_type=jax.ShapeDtypeStruct((_V, 128), jnp.float32),
        mesh=_mesh(),
        scratch_types=[
            pltpu.VMEM((2, _RCH, _D), jnp.float32),
            pltpu.VMEM((2, _RCH, 128), jnp.float32),
            pltpu.SemaphoreType.DMA((2,)),
            pltpu.SemaphoreType.DMA((2,)),
        ],
        compiler_params=pltpu.CompilerParams(use_tc_tiling_on_sc=True, needs_layout_passes=False),
    )
    def conv(w_hbm, w2_hbm, a_v, b_v, rsem, wsem):
        wid = lax.axis_index("s") * 2 + lax.axis_index("c")
        r0 = jnp.minimum(wid * _RW, _V - _RW)

        def fire_r(c, sl):
            pltpu.async_copy(
                w_hbm.at[pl.ds(r0 + _RCH * c, _RCH)], a_v.at[sl], rsem.at[sl]
            )

        def drain_r(c, sl):
            pltpu.make_async_copy(
                w_hbm.at[pl.ds(r0 + _RCH * c, _RCH)], a_v.at[sl], rsem.at[sl]
            ).wait()

        def fire_w(c, sl):
            pltpu.async_copy(
                b_v.at[sl], w2_hbm.at[pl.ds(r0 + _RCH * c, _RCH)], wsem.at[sl]
            )

        def drain_w(c, sl):
            pltpu.make_async_copy(
                b_v.at[sl], w2_hbm.at[pl.ds(r0 + _RCH * c, _RCH)], wsem.at[sl]
            ).wait()

        def tec(sl):
            def row(q, _):
                for k0 in range(0, _D, 16):
                    b_v[sl, q, pl.ds(k0, 16)] = a_v[sl, q, pl.ds(k0, 16)]
                return 0

            lax.fori_loop(0, _RCH, row, 0)

        fire_r(0, 0)
        fire_r(1, 1)
        for c in (0, 1):
            drain_r(c, c)
            tec(c)
            fire_w(c, c)
            fire_r(c + 2, c)

        def body(c, _):
            sl = lax.rem(c, 2)
            drain_r(c, sl)
            drain_w(c - 2, sl)
            tec(sl)
            fire_w(c, sl)
            fire_r(c + 2, sl)
            return 0

        lax.fori_loop(2, _NCH_W - 2, body, 0)

        for c in (_NCH_W - 2, _NCH_W - 1):
            sl = c % 2
            drain_r(c, sl)
            drain_w(c - 2, sl)
            tec(sl)
            fire_w(c, sl)
        drain_w(_NCH_W - 2, (_NCH_W - 2) % 2)
        drain_w(_NCH_W - 1, (_NCH_W - 1) % 2)

    return conv(weight)


@functools.partial(jax.jit, static_argnums=(2, 3))
def _gather(x32, w2, b, s):
    @functools.partial(
        pl.kernel,
        out_type=jax.ShapeDtypeStruct((s, _D, b), jnp.float32),
        mesh=_mesh(),
        scratch_types=[
            pltpu.VMEM((_XR_W, s), jnp.int32),
            pltpu.VMEM((s, _XR_W), jnp.int32),
            pltpu.VMEM((2, _SCH, _XR_W, 128), jnp.float32),
            pltpu.VMEM((2, _SCH, _D, _XR_W), jnp.float32),
            pltpu.SemaphoreType.DMA((2,)),
            pltpu.SemaphoreType.DMA((2,)),
        ],
        compiler_params=pltpu.CompilerParams(use_tc_tiling_on_sc=True, needs_layout_passes=False),
    )
    def gath(w2_hbm, x_hbm, out_hbm, idx_v, idxt_v, pair_v, stage_v, gsem,
             ssem):
        wid = lax.axis_index("s") * 2 + lax.axis_index("c")
        bx = wid * _XR_W
        pltpu.sync_copy(x_hbm.at[pl.ds(bx, _XR_W)], idx_v)

        lanes = lax.iota(jnp.int32, 16)

        def trow(t, _):
            for b0 in range(0, _XR_W, 16):
                v = plsc.load_gather(
                    idx_v, [lanes + b0, jnp.full((16,), 0, jnp.int32) + t]
                )
                idxt_v[t, pl.ds(b0, 16)] = v
            return 0

        lax.fori_loop(0, s, trow, 0)

        def fire_g(c, sl):
            for j in range(_SCH):
                pltpu.async_copy(
                    w2_hbm.at[idxt_v.at[_SCH * c + j]],
                    pair_v.at[sl].at[j],
                    gsem.at[sl],
                )

        def drain_g(c, sl):
            for j in range(_SCH):
                pltpu.make_async_copy(
                    w2_hbm.at[idxt_v.at[_SCH * c + j]],
                    pair_v.at[sl].at[j],
                    gsem.at[sl],
                ).wait()

        def transpose(sl):
            def td(d, _):
                for j in range(_SCH):
                    for b0 in range(0, _XR_W, 16):
                        v = plsc.load_gather(
                            pair_v,
                            [
                                jnp.full((16,), sl, jnp.int32),
                                jnp.full((16,), j, jnp.int32),
                                lanes + b0,
                                jnp.full((16,), 0, jnp.int32) + d,
                            ],
                        )
                        stage_v[sl, j, d, pl.ds(b0, 16)] = v
                return 0

            lax.fori_loop(0, _D, td, 0)

        def fire_w(c, sl):
            pltpu.async_copy(
                stage_v.at[sl],
                out_hbm.at[pl.ds(_SCH * c, _SCH), slice(None),
                           pl.ds(bx, _XR_W)],
                ssem.at[sl],
            )

        def drain_w(c, sl):
            pltpu.make_async_copy(
                stage_v.at[sl],
                out_hbm.at[pl.ds(_SCH * c, _SCH), slice(None),
                           pl.ds(bx, _XR_W)],
                ssem.at[sl],
            ).wait()

        fire_g(0, 0)
        fire_g(1, 1)
        for c in (0, 1):
            drain_g(c, c)
            transpose(c)
            fire_w(c, c)
            fire_g(c + 2, c)

        def body(i, _):
            for sl in (0, 1):
                c = 2 * i + sl
                drain_g(c, sl)
                drain_w(c - 2, sl)
                transpose(sl)
                fire_w(c, sl)
                fire_g(c + 2, sl)
            return 0

        lax.fori_loop(1, 11, body, 0)

        c = 22
        drain_g(c, 0)
        drain_w(20, 0)
        transpose(0)
        fire_w(c, 0)
        fire_g(24, 0)
        c = 23
        drain_g(c, 1)
        drain_w(21, 1)
        transpose(1)
        fire_w(c, 1)
        c = 24
        drain_g(c, 0)
        drain_w(22, 0)
        transpose(0)
        fire_w(c, 0)
        drain_w(23, 1)
        drain_w(24, 0)

    return gath(w2, x32)


def kernel(x, weight):
    b, s = x.shape
    w2 = _widen(weight)
    out_t = _gather(x.astype(jnp.int32), w2, b, s)
    return jnp.transpose(out_t, (2, 0, 1))


# speedup vs baseline: 1.1677x; 1.1677x over previous
import functools

import jax
import jax.numpy as jnp
from jax import lax
from jax.experimental import pallas as pl
from jax.experimental.pallas import tpu as pltpu
from jax.experimental.pallas import tpu_sc as plsc

_D = 64
_NW = 32
_V = 100000
_RW = 3200
_RCH = 160
_NCH_W = _RW // _RCH
_XR_W = 128
_SCH = 2
_NCH_G = 50 // _SCH


def _mesh():
    return plsc.VectorSubcoreMesh(core_axis_name="c", subcore_axis_name="s")


@jax.jit
def _widen(weight):
    @functools.partial(
        pl.kernel,
        out_type=jax.ShapeDtypeStruct((_V, 128), jnp.float32),
        mesh=_mesh(),
        scratch_types=[
            pltpu.VMEM((2, _RCH, _D), jnp.float32),
            pltpu.VMEM((2, _RCH, 128), jnp.float32),
            pltpu.SemaphoreType.DMA((2,)),
            pltpu.SemaphoreType.DMA((2,)),
        ],
        compiler_params=pltpu.CompilerParams(use_tc_tiling_on_sc=True, needs_layout_passes=False),
    )
    def conv(w_hbm, w2_hbm, a_v, b_v, rsem, wsem):
        wid = lax.axis_index("s") * 2 + lax.axis_index("c")
        r0 = jnp.minimum(wid * _RW, _V - _RW)

        def fire_r(c, sl):
            pltpu.async_copy(
                w_hbm.at[pl.ds(r0 + _RCH * c, _RCH)], a_v.at[sl], rsem.at[sl]
            )

        def drain_r(c, sl):
            pltpu.make_async_copy(
                w_hbm.at[pl.ds(r0 + _RCH * c, _RCH)], a_v.at[sl], rsem.at[sl]
            ).wait()

        def fire_w(c, sl):
            pltpu.async_copy(
                b_v.at[sl], w2_hbm.at[pl.ds(r0 + _RCH * c, _RCH)], wsem.at[sl]
            )

        def drain_w(c, sl):
            pltpu.make_async_copy(
                b_v.at[sl], w2_hbm.at[pl.ds(r0 + _RCH * c, _RCH)], wsem.at[sl]
            ).wait()

        def tec(sl):
            def row4(q4, _):
                for u in range(4):
                    q = q4 * 4 + u
                    for k0 in range(0, _D, 16):
                        b_v[sl, q, pl.ds(k0, 16)] = a_v[sl, q, pl.ds(k0, 16)]
                return 0

            lax.fori_loop(0, _RCH // 4, row4, 0)

        fire_r(0, 0)
        fire_r(1, 1)
        for c in (0, 1):
            drain_r(c, c)
            tec(c)
            fire_w(c, c)
            fire_r(c + 2, c)

        def body(c, _):
            sl = lax.rem(c, 2)
            drain_r(c, sl)
            drain_w(c - 2, sl)
            tec(sl)
            fire_w(c, sl)
            fire_r(c + 2, sl)
            return 0

        lax.fori_loop(2, _NCH_W - 2, body, 0)

        for c in (_NCH_W - 2, _NCH_W - 1):
            sl = c % 2
            drain_r(c, sl)
            drain_w(c - 2, sl)
            tec(sl)
            fire_w(c, sl)
        drain_w(_NCH_W - 2, (_NCH_W - 2) % 2)
        drain_w(_NCH_W - 1, (_NCH_W - 1) % 2)

    return conv(weight)


@functools.partial(jax.jit, static_argnums=(2, 3))
def _gather(x32, w2, b, s):
    @functools.partial(
        pl.kernel,
        out_type=jax.ShapeDtypeStruct((s, _D, b), jnp.float32),
        mesh=_mesh(),
        scratch_types=[
            pltpu.VMEM((_XR_W, s), jnp.int32),
            pltpu.VMEM((s, _XR_W), jnp.int32),
            pltpu.VMEM((2, _SCH, _XR_W, 128), jnp.float32),
            pltpu.VMEM((2, _SCH * _D, _XR_W), jnp.float32),
            pltpu.SemaphoreType.DMA((2,)),
            pltpu.SemaphoreType.DMA((2,)),
        ],
        compiler_params=pltpu.CompilerParams(use_tc_tiling_on_sc=True, needs_layout_passes=False),
    )
    def gath(w2_hbm, x_hbm, out_hbm, idx_v, idxt_v, pair_v, stage_v, gsem,
             ssem):
        wid = lax.axis_index("s") * 2 + lax.axis_index("c")
        bx = wid * _XR_W
        pltpu.sync_copy(x_hbm.at[pl.ds(bx, _XR_W)], idx_v)

        lanes = lax.iota(jnp.int32, 16)

        def trow(t, _):
            for b0 in range(0, _XR_W, 16):
                v = plsc.load_gather(
                    idx_v, [lanes + b0, jnp.full((16,), 0, jnp.int32) + t]
                )
                idxt_v[t, pl.ds(b0, 16)] = v
            return 0

        lax.fori_loop(0, s, trow, 0)

        def fire_g(c, sl):
            for j in range(_SCH):
                pltpu.async_copy(
                    w2_hbm.at[idxt_v.at[_SCH * c + j]],
                    pair_v.at[sl].at[j],
                    gsem.at[sl],
                )

        def drain_g(c, sl):
            for j in range(_SCH):
                pltpu.make_async_copy(
                    w2_hbm.at[idxt_v.at[_SCH * c + j]],
                    pair_v.at[sl].at[j],
                    gsem.at[sl],
                ).wait()

        rvecs = [
            lanes + (j * _D + d0)
            for j in range(_SCH)
            for d0 in range(0, _D, 16)
        ]

        def transpose(sl):
            sref = stage_v.at[sl]

            def tb(b, _):
                cvec = jnp.full((16,), 0, jnp.int32) + b
                for j in range(_SCH):
                    for di, d0 in enumerate(range(0, _D, 16)):
                        v = pair_v[sl, j, b, pl.ds(d0, 16)]
                        plsc.store_scatter(
                            sref, [rvecs[j * 4 + di], cvec], v
                        )
                return 0

            lax.fori_loop(0, _XR_W, tb, 0)

        def fire_w(c, sl):
            for j in range(_SCH):
                pltpu.async_copy(
                    stage_v.at[sl].at[pl.ds(j * _D, _D)],
                    out_hbm.at[_SCH * c + j].at[slice(None),
                                                pl.ds(bx, _XR_W)],
                    ssem.at[sl],
                )

        def drain_w(c, sl):
            for j in range(_SCH):
                pltpu.make_async_copy(
                    stage_v.at[sl].at[pl.ds(j * _D, _D)],
                    out_hbm.at[_SCH * c + j].at[slice(None),
                                                pl.ds(bx, _XR_W)],
                    ssem.at[sl],
                ).wait()

        fire_g(0, 0)
        fire_g(1, 1)
        for c in (0, 1):
            drain_g(c, c)
            transpose(c)
            fire_w(c, c)
            fire_g(c + 2, c)

        def body(i, _):
            for sl in (0, 1):
                c = 2 * i + sl
                drain_g(c, sl)
                drain_w(c - 2, sl)
                transpose(sl)
                fire_w(c, sl)
                fire_g(c + 2, sl)
            return 0

        lax.fori_loop(1, 11, body, 0)

        c = 22
        drain_g(c, 0)
        drain_w(20, 0)
        transpose(0)
        fire_w(c, 0)
        fire_g(24, 0)
        c = 23
        drain_g(c, 1)
        drain_w(21, 1)
        transpose(1)
        fire_w(c, 1)
        c = 24
        drain_g(c, 0)
        drain_w(22, 0)
        transpose(0)
        fire_w(c, 0)
        drain_w(23, 1)
        drain_w(24, 0)

    return gath(w2, x32)


def kernel(x, weight):
    b, s = x.shape
    w2 = _widen(weight)
    out_t = _gather(x.astype(jnp.int32), w2, b, s)
    return jnp.transpose(out_t, (2, 0, 1))


# speedup vs baseline: 1.4930x; 1.2786x over previous
import functools

import jax
import jax.numpy as jnp
from jax import lax
from jax.experimental import pallas as pl
from jax.experimental.pallas import tpu as pltpu
from jax.experimental.pallas import tpu_sc as plsc

_D = 64
_NW = 32
_V = 100000
_RW = 3200
_RCH = 160
_NCH_W = _RW // _RCH
_SPLIT = 2
_XCH = 4


def _mesh():
    return plsc.VectorSubcoreMesh(core_axis_name="c", subcore_axis_name="s")


@jax.jit
def _widen(weight):
    @functools.partial(
        pl.kernel,
        out_type=jax.ShapeDtypeStruct((_V, 128), jnp.float32),
        mesh=_mesh(),
        scratch_types=[
            pltpu.VMEM((2, _RCH, _D), jnp.float32),
            pltpu.VMEM((2, _RCH, 128), jnp.float32),
            pltpu.SemaphoreType.DMA((2,)),
            pltpu.SemaphoreType.DMA((2,)),
        ],
        compiler_params=pltpu.CompilerParams(use_tc_tiling_on_sc=True),
    )
    def conv(w_hbm, w2_hbm, a_v, b_v, rsem, wsem):
        wid = lax.axis_index("s") * 2 + lax.axis_index("c")
        r0 = jnp.minimum(wid * _RW, _V - _RW)

        def fire_r(c, sl):
            pltpu.async_copy(
                w_hbm.at[pl.ds(r0 + _RCH * c, _RCH)], a_v.at[sl], rsem.at[sl]
            )

        def drain_r(c, sl):
            pltpu.make_async_copy(
                w_hbm.at[pl.ds(r0 + _RCH * c, _RCH)], a_v.at[sl], rsem.at[sl]
            ).wait()

        def fire_w(c, sl):
            pltpu.async_copy(
                b_v.at[sl], w2_hbm.at[pl.ds(r0 + _RCH * c, _RCH)], wsem.at[sl]
            )

        def drain_w(c, sl):
            pltpu.make_async_copy(
                b_v.at[sl], w2_hbm.at[pl.ds(r0 + _RCH * c, _RCH)], wsem.at[sl]
            ).wait()

        def tec(sl):
            def row4(q4, _):
                for u in range(4):
                    q = q4 * 4 + u
                    for k0 in range(0, _D, 16):
                        b_v[sl, q, pl.ds(k0, 16)] = a_v[sl, q, pl.ds(k0, 16)]
                return 0

            lax.fori_loop(0, _RCH // 4, row4, 0)

        fire_r(0, 0)
        fire_r(1, 1)
        for c in (0, 1):
            drain_r(c, c)
            tec(c)
            fire_w(c, c)
            fire_r(c + 2, c)

        def body(c, _):
            sl = lax.rem(c, 2)
            drain_r(c, sl)
            drain_w(c - 2, sl)
            tec(sl)
            fire_w(c, sl)
            fire_r(c + 2, sl)
            return 0

        lax.fori_loop(2, _NCH_W - 2, body, 0)

        for c in (_NCH_W - 2, _NCH_W - 1):
            sl = c % 2
            drain_r(c, sl)
            drain_w(c - 2, sl)
            tec(sl)
            fire_w(c, sl)
        drain_w(_NCH_W - 2, (_NCH_W - 2) % 2)
        drain_w(_NCH_W - 1, (_NCH_W - 1) % 2)

    return conv(weight)


@functools.partial(jax.jit, static_argnums=(2, 3, 4))
def _gather(x32, w2, bh, s, b0):
    xr_w = bh // _NW
    nch = xr_w // _XCH

    @functools.partial(
        pl.kernel,
        out_type=jax.ShapeDtypeStruct((bh, s, _D), jnp.float32),
        mesh=_mesh(),
        scratch_types=[
            pltpu.VMEM((xr_w, s), jnp.int32),
            pltpu.VMEM((2, _XCH, s, 128), jnp.float32),
            pltpu.VMEM((2, _XCH, s, _D), jnp.float32),
            pltpu.SemaphoreType.DMA((2,)),
            pltpu.SemaphoreType.DMA((2,)),
        ],
        compiler_params=pltpu.CompilerParams(use_tc_tiling_on_sc=True),
    )
    def gath(w2_hbm, x_hbm, out_hbm, idx_v, pair_v, stage_v, gsem, ssem):
        wid = lax.axis_index("s") * 2 + lax.axis_index("c")
        bx = wid * xr_w
        pltpu.sync_copy(x_hbm.at[pl.ds(b0 + bx, xr_w)], idx_v)

        def fire_g(c, sl):
            for j in range(_XCH):
                pltpu.async_copy(
                    w2_hbm.at[idx_v.at[_XCH * c + j]],
                    pair_v.at[sl].at[j],
                    gsem.at[sl],
                )

        def drain_g(c, sl):
            for j in range(_XCH):
                pltpu.make_async_copy(
                    w2_hbm.at[idx_v.at[_XCH * c + j]],
                    pair_v.at[sl].at[j],
                    gsem.at[sl],
                ).wait()

        def extract(sl):
            def ej(j, _):
                def et(t, _):
                    for k0 in range(0, _D, 16):
                        stage_v[sl, j, t, pl.ds(k0, 16)] = pair_v[
                            sl, j, t, pl.ds(k0, 16)
                        ]
                    return 0

                lax.fori_loop(0, s, et, 0)
                return 0

            lax.fori_loop(0, _XCH, ej, 0)

        def fire_w(c, sl):
            pltpu.async_copy(
                stage_v.at[sl],
                out_hbm.at[pl.ds(bx + _XCH * c, _XCH)],
                ssem.at[sl],
            )

        def drain_w(c, sl):
            pltpu.make_async_copy(
                stage_v.at[sl],
                out_hbm.at[pl.ds(bx + _XCH * c, _XCH)],
                ssem.at[sl],
            ).wait()

        fire_g(0, 0)
        fire_g(1, 1)
        for c in (0, 1):
            drain_g(c, c)
            extract(c)
            fire_w(c, c)
            fire_g(c + 2, c)

        def body(i, _):
            for sl in (0, 1):
                c = 2 * i + sl
                drain_g(c, sl)
                drain_w(c - 2, sl)
                extract(sl)
                fire_w(c, sl)
                fire_g(c + 2, sl)
            return 0

        lax.fori_loop(1, nch // 2 - 1, body, 0)

        for c in (nch - 2, nch - 1):
            sl = c % 2
            drain_g(c, sl)
            drain_w(c - 2, sl)
            extract(sl)
            fire_w(c, sl)
        drain_w(nch - 2, nch % 2)
        drain_w(nch - 1, (nch + 1) % 2)

    return gath(w2, x32)


def kernel(x, weight):
    b, s = x.shape
    w2 = _widen(weight)
    x32 = x.astype(jnp.int32)
    bh = b // _SPLIT
    halves = [
        _gather(x32, w2, bh, s, i * bh) for i in range(_SPLIT)
    ]
    return jnp.concatenate(halves, axis=0)


# speedup vs baseline: 1.6199x; 1.0849x over previous
import functools

import jax
import jax.numpy as jnp
from jax import lax
from jax.experimental import pallas as pl
from jax.experimental.pallas import tpu as pltpu
from jax.experimental.pallas import tpu_sc as plsc

_D = 64
_NW = 32
_XROWS_PER_CHUNK = 8
_NSLOT = 4


@functools.partial(jax.jit, static_argnums=(2, 3, 4))
def _sc_embedding_gather(x32, weight, bh, s, b0):
    xrows_per_w = bh // _NW
    n_chunks = xrows_per_w // _XROWS_PER_CHUNK
    mesh = plsc.VectorSubcoreMesh(core_axis_name="c", subcore_axis_name="s")

    @functools.partial(
        pl.kernel,
        out_type=jax.ShapeDtypeStruct((bh, s, _D), jnp.float32),
        mesh=mesh,
        scratch_types=[
            pltpu.VMEM((xrows_per_w, s), jnp.int32),
            pltpu.VMEM((_NSLOT, _XROWS_PER_CHUNK, s, _D), jnp.float32),
            pltpu.SemaphoreType.DMA((_NSLOT,)),
            pltpu.SemaphoreType.DMA((_NSLOT,)),
        ],
        compiler_params=pltpu.CompilerParams(use_tc_tiling_on_sc=False),
    )
    def k(table_hbm, x_hbm, out_hbm, idx_v, rows_v, gsem, ssem):
        wid = lax.axis_index("s") * 2 + lax.axis_index("c")
        base_x = wid * xrows_per_w
        pltpu.sync_copy(x_hbm.at[pl.ds(b0 + base_x, xrows_per_w)], idx_v)

        def fire_gathers(c, slot):
            ds = []
            for j in range(_XROWS_PER_CHUNK):
                ds.append(
                    pltpu.async_copy(
                        table_hbm.at[idx_v.at[c * _XROWS_PER_CHUNK + j]],
                        rows_v.at[slot].at[j],
                        gsem.at[slot],
                    )
                )
            return ds

        def drain_gathers(c, slot):
            for j in range(_XROWS_PER_CHUNK):
                pltpu.make_async_copy(
                    table_hbm.at[idx_v.at[c * _XROWS_PER_CHUNK + j]],
                    rows_v.at[slot].at[j],
                    gsem.at[slot],
                ).wait()

        def fire_scatter(c, slot):
            return pltpu.async_copy(
                rows_v.at[slot],
                out_hbm.at[pl.ds(base_x + c * _XROWS_PER_CHUNK, _XROWS_PER_CHUNK)],
                ssem.at[slot],
            )

        def drain_scatter(c, slot):
            pltpu.make_async_copy(
                rows_v.at[slot],
                out_hbm.at[pl.ds(base_x + c * _XROWS_PER_CHUNK, _XROWS_PER_CHUNK)],
                ssem.at[slot],
            ).wait()

        for c in range(_NSLOT):
            fire_gathers(c, c)

        def body(c, _):
            slot = lax.rem(c, _NSLOT)
            drain_gathers(c, slot)
            fire_scatter(c, slot)
            drain_scatter(c, slot)
            fire_gathers(c + _NSLOT, slot)
            return 0

        lax.fori_loop(0, n_chunks - _NSLOT, body, 0)

        for c in range(n_chunks - _NSLOT, n_chunks):
            slot = c % _NSLOT
            drain_gathers(c, slot)
            fire_scatter(c, slot)
        for c in range(n_chunks - _NSLOT, n_chunks):
            drain_scatter(c, c % _NSLOT)

    return k(weight, x32)


def kernel(x, weight):
    b, s = x.shape
    x32 = x.astype(jnp.int32)
    bh = b // 2
    halves = [
        _sc_embedding_gather(x32, weight, bh, s, i * bh) for i in range(2)
    ]
    return jnp.concatenate(halves, axis=0)


# speedup vs baseline: 1.7414x; 1.0750x over previous
import functools

import jax
import jax.numpy as jnp
from jax import lax
from jax.experimental import pallas as pl
from jax.experimental.pallas import tpu as pltpu
from jax.experimental.pallas import tpu_sc as plsc

_D = 64
_NW = 32
_ROWS_PER_CHUNK = 640
_NSLOT = 3


@functools.partial(jax.jit, static_argnums=(2,))
def _sc_embedding_gather(idx2d, weight, b_total):
    b_per_w = b_total // _NW
    n_chunks = b_per_w // _ROWS_PER_CHUNK
    mesh = plsc.VectorSubcoreMesh(core_axis_name="c", subcore_axis_name="s")

    @functools.partial(
        pl.kernel,
        out_type=jax.ShapeDtypeStruct((b_total, _D), jnp.float32),
        mesh=mesh,
        scratch_types=[
            pltpu.VMEM((b_per_w,), jnp.int32),
            pltpu.VMEM((_NSLOT, _ROWS_PER_CHUNK, _D), jnp.float32),
            pltpu.SemaphoreType.DMA((_NSLOT,)),
            pltpu.SemaphoreType.DMA((_NSLOT,)),
        ],
        compiler_params=pltpu.CompilerParams(use_tc_tiling_on_sc=False),
    )
    def k(table_hbm, idx_hbm, out_hbm, idx_v, rows_v, gsem, ssem):
        wid = lax.axis_index("s") * 2 + lax.axis_index("c")
        base = wid * b_per_w
        pltpu.sync_copy(idx_hbm.at[wid], idx_v)

        def fire_gather(c, slot):
            return pltpu.async_copy(
                table_hbm.at[idx_v.at[pl.ds(c * _ROWS_PER_CHUNK, _ROWS_PER_CHUNK)]],
                rows_v.at[slot],
                gsem.at[slot],
            )

        def fire_scatter(c, slot):
            return pltpu.async_copy(
                rows_v.at[slot],
                out_hbm.at[pl.ds(base + c * _ROWS_PER_CHUNK, _ROWS_PER_CHUNK)],
                ssem.at[slot],
            )

        gd = {}
        sd = {}
        for c in range(_NSLOT):
            gd[c] = fire_gather(c, c % _NSLOT)
        for c in range(n_chunks):
            slot = c % _NSLOT
            gd[c].wait()
            sd[c] = fire_scatter(c, slot)
            nxt = c + _NSLOT
            if nxt < n_chunks:
                sd[c].wait()
                gd[nxt] = fire_gather(nxt, slot)
        for c in range(n_chunks - _NSLOT, n_chunks):
            sd[c].wait()

    return k(weight, idx2d)


def kernel(x, weight):
    b, s = x.shape
    b_total = b * s
    idx2d = x.reshape(_NW, b_total // _NW).astype(jnp.int32)
    out = _sc_embedding_gather(idx2d, weight, b_total)
    return out.reshape(b, s, _D)
